# Initial kernel scaffold; baseline (speedup 1.0000x reference)
import jax
import jax.numpy as jnp
from jax.experimental import pallas as pl


def kernel(x, edge_index, edge_weight, W1, b1, W2, b2):
    raise NotImplementedError("write your pallas kernel here")


# speedup vs baseline: 8.3298x; 8.3298x over previous
import functools

import jax
import jax.numpy as jnp
from jax import lax
from jax.experimental import pallas as pl
from jax.experimental.pallas import tpu as pltpu
from jax.experimental.pallas import tpu_sc as plsc

NC = 2
NS = 16
NW = NC * NS
L = 16

K = 80


def _worker(base_per_worker):
    cid = lax.axis_index("c")
    sid = lax.axis_index("s")
    wid = cid * NS + sid
    return cid, sid, wid * base_per_worker


def _deg_body(nnodes, epw, dst_hbm, ew_hbm, zero_hbm, deg_out, dst2, ewv, rep,
              degsh):
    nps = nnodes // NS
    cid, sid, base = _worker(epw)
    pltpu.sync_copy(zero_hbm.at[pl.ds(sid * nps, nps)],
                    degsh.at[pl.ds(sid * nps, nps)])
    plsc.subcore_barrier()

    nchunk = epw // K

    def chunk(c, _):
        cb = base + c * K
        pltpu.sync_copy(dst_hbm.at[pl.ds(cb, K)], dst2.at[0])
        pltpu.sync_copy(ew_hbm.at[pl.ds(cb, K)], ewv)

        def build(e, _):
            idx = jnp.zeros((L,), jnp.int32) + e
            rep[e, :] = plsc.load_gather(ewv, [idx])
            return 0

        lax.fori_loop(0, K, build, 0)
        pltpu.sync_copy(rep, degsh.at[dst2.at[0]], add=True)
        return 0

    lax.fori_loop(0, nchunk, chunk, 0)
    plsc.subcore_barrier()
    pltpu.sync_copy(degsh.at[pl.ds(sid * nps, nps)],
                    deg_out.at[cid, pl.ds(sid * nps, nps)])


def _norm_body(nnodes, epw, src_hbm, dst_hbm, ew_hbm, dinv_hbm, norm_out,
               srcv, dstv, ewv, dinvv, normv):
    _, _, base = _worker(epw)
    pltpu.sync_copy(dinv_hbm, dinvv)
    pltpu.sync_copy(src_hbm.at[pl.ds(base, epw)], srcv)
    pltpu.sync_copy(dst_hbm.at[pl.ds(base, epw)], dstv)
    pltpu.sync_copy(ew_hbm.at[pl.ds(base, epw)], ewv)

    def step(i, _):
        sl = pl.ds(i * L, L)
        ds = plsc.load_gather(dinvv, [srcv[sl]])
        dd = plsc.load_gather(dinvv, [dstv[sl]])
        normv[sl] = ds * ewv[sl] * dd
        return 0

    lax.fori_loop(0, epw // L, step, 0)
    pltpu.sync_copy(normv, norm_out.at[pl.ds(base, epw)])


def _agg_body(nnodes, d, epw, h_hbm, src_hbm, dst_hbm, norm_hbm, zero_hbm,
              out_hbm, idxv, dst2, normv, rows, accsh, gsem):
    nps = nnodes // NS
    cid, sid, base = _worker(epw)
    pltpu.sync_copy(zero_hbm.at[pl.ds(sid * nps, nps)],
                    accsh.at[pl.ds(sid * nps, nps)])
    plsc.subcore_barrier()

    nchunk = epw // K
    ncol = d // L

    def chunk(c, _):
        cb = base + c * K
        pltpu.sync_copy(src_hbm.at[pl.ds(cb, K)], idxv)
        pltpu.sync_copy(dst_hbm.at[pl.ds(cb, K)], dst2.at[0])
        pltpu.sync_copy(norm_hbm.at[pl.ds(cb, K)], normv)
        pltpu.async_copy(h_hbm.at[idxv], rows, gsem).wait()

        def scale(e, _):
            nv = plsc.load_gather(normv, [jnp.zeros((L,), jnp.int32) + e])
            for j in range(ncol):
                sl = pl.ds(j * L, L)
                rows[e, sl] = rows[e, sl] * nv
            return 0

        lax.fori_loop(0, K, scale, 0)
        pltpu.sync_copy(rows, accsh.at[dst2.at[0]], add=True)
        return 0

    lax.fori_loop(0, nchunk, chunk, 0)
    plsc.subcore_barrier()
    pltpu.sync_copy(accsh.at[pl.ds(sid * nps, nps)],
                    out_hbm.at[cid, pl.ds(sid * nps, nps)])


def _dinv_body(degp_ref, dinv_ref):
    deg = degp_ref[0, :, 0:1] + degp_ref[1, :, 0:1] + 1.0
    dinv_ref[:] = lax.rsqrt(deg)


def _mm_body(x_ref, w_ref, o_ref):
    o_ref[:] = jnp.dot(x_ref[:], w_ref[:], preferred_element_type=jnp.float32)


def _combine_body(a0_ref, a1_ref, h_ref, dinv_ref, b_ref, o_ref):
    di = dinv_ref[:]
    o_ref[:] = jax.nn.relu(a0_ref[:] + a1_ref[:] + (di * di) * h_ref[:]
                           + b_ref[:])


def _combine_mm_body(a0_ref, a1_ref, h_ref, dinv_ref, b_ref, w_ref, o_ref):
    di = dinv_ref[:]
    o = jax.nn.relu(a0_ref[:] + a1_ref[:] + (di * di) * h_ref[:] + b_ref[:])
    o_ref[:] = jnp.dot(o, w_ref[:], preferred_element_type=jnp.float32)


def _sc_mesh():
    return plsc.VectorSubcoreMesh(core_axis_name="c", subcore_axis_name="s",
                                  num_cores=NC, num_subcores=NS)


_SC_PARAMS = pltpu.CompilerParams(use_tc_tiling_on_sc=False,
                                  needs_layout_passes=False)


def kernel(x, edge_index, edge_weight, W1, b1, W2, b2):
    n, d = x.shape
    e = edge_index.shape[1]
    epw = e // NW
    npad = ((n + NS * 8 - 1) // (NS * 8)) * (NS * 8)
    assert e % (NW * K) == 0 and d % L == 0

    src = edge_index[0].astype(jnp.int32)
    dst = edge_index[1].astype(jnp.int32)
    ew = edge_weight.astype(jnp.float32)
    zeros_nd = jnp.zeros((npad, d), jnp.float32)
    zeros_n16 = jnp.zeros((npad, L), jnp.float32)

    f32 = jnp.float32

    deg_part = pl.kernel(
        functools.partial(_deg_body, npad, epw),
        out_type=jax.ShapeDtypeStruct((NC, npad, L), f32),
        mesh=_sc_mesh(),
        compiler_params=_SC_PARAMS,
        scratch_types=[
            pltpu.VMEM((1, K), jnp.int32),
            pltpu.VMEM((K,), f32),
            pltpu.VMEM((K, L), f32),
            pltpu.VMEM_SHARED((npad, L), f32),
        ],
    )(dst, ew, zeros_n16)

    dinv_pad = pl.pallas_call(
        _dinv_body,
        out_shape=jax.ShapeDtypeStruct((npad, 1), f32),
    )(deg_part)
    dinv_flat = dinv_pad[:, 0]
    dinv = dinv_pad[:n]

    norm = pl.kernel(
        functools.partial(_norm_body, n, epw),
        out_type=jax.ShapeDtypeStruct((e,), f32),
        mesh=_sc_mesh(),
        compiler_params=_SC_PARAMS,
        scratch_types=[
            pltpu.VMEM((epw,), jnp.int32),
            pltpu.VMEM((epw,), jnp.int32),
            pltpu.VMEM((epw,), f32),
            pltpu.VMEM((npad,), f32),
            pltpu.VMEM((epw,), f32),
        ],
    )(src, dst, ew, dinv_flat)

    bm = 2000
    grid = n // bm
    row_spec = pl.BlockSpec((bm, d), lambda i: (i, 0))
    col1_spec = pl.BlockSpec((bm, 1), lambda i: (i, 0))
    w_spec = pl.BlockSpec((d, d), lambda i: (0, 0))
    b_spec = pl.BlockSpec((1, d), lambda i: (0, 0))
    out_nd = jax.ShapeDtypeStruct((n, d), f32)

    def sc_agg(h):
        return pl.kernel(
            functools.partial(_agg_body, npad, d, epw),
            out_type=jax.ShapeDtypeStruct((NC, npad, d), f32),
            mesh=_sc_mesh(),
            compiler_params=_SC_PARAMS,
            scratch_types=[
                pltpu.VMEM((K,), jnp.int32),
                pltpu.VMEM((1, K), jnp.int32),
                pltpu.VMEM((K,), f32),
                pltpu.VMEM((K, d), f32),
                pltpu.VMEM_SHARED((npad, d), f32),
                pltpu.SemaphoreType.DMA,
            ],
        )(h, src, dst, norm, zeros_nd)

    h1 = pl.pallas_call(
        _mm_body, grid=(grid,),
        in_specs=[row_spec, w_spec],
        out_specs=row_spec,
        out_shape=out_nd,
    )(x, W1)

    aggp = sc_agg(h1)
    agg1 = aggp[:, :n]

    h2 = pl.pallas_call(
        _combine_mm_body, grid=(grid,),
        in_specs=[row_spec, row_spec, row_spec, col1_spec, b_spec, w_spec],
        out_specs=row_spec,
        out_shape=out_nd,
    )(agg1[0], agg1[1], h1, dinv, b1.reshape(1, d), W2)

    aggp2 = sc_agg(h2)
    agg2 = aggp2[:, :n]

    out = pl.pallas_call(
        _combine_body, grid=(grid,),
        in_specs=[row_spec, row_spec, row_spec, col1_spec, b_spec],
        out_specs=row_spec,
        out_shape=out_nd,
    )(agg2[0], agg2[1], h2, dinv, b2.reshape(1, d))
    return out
